# baseline (speedup 1.0000x reference)
import functools

import jax
import jax.numpy as jnp
from jax.experimental import pallas as pl
from jax.experimental.pallas import tpu as pltpu

N = 10000
C = 128
BM = 400


def _lp_kernel(y_ref, mask_ref, prop_ref, out_ref, label_ref):
    @pl.when(pl.program_id(0) == 0)
    def _build_label():
        classes = jax.lax.broadcasted_iota(jnp.int32, (N, C), 1)
        eq = classes == y_ref[:]
        label_ref[:] = jnp.where(eq, mask_ref[:], 0.0).astype(jnp.bfloat16)

    acc = jax.lax.dot_general(
        prop_ref[:].astype(jnp.bfloat16),
        label_ref[:],
        (((1,), (0,)), ((), ())),
        preferred_element_type=jnp.float32,
    )
    out_ref[:] = jnp.clip(acc, 0.0, 1.0)


@functools.partial(jax.jit, static_argnames=())
def kernel(x, y, train_mask, prop):
    del x
    y2 = y.reshape(N, 1)
    mask2 = train_mask.astype(jnp.float32).reshape(N, 1)
    return pl.pallas_call(
        _lp_kernel,
        grid=(N // BM,),
        in_specs=[
            pl.BlockSpec((N, 1), lambda i: (0, 0)),
            pl.BlockSpec((N, 1), lambda i: (0, 0)),
            pl.BlockSpec((BM, N), lambda i: (i, 0)),
        ],
        out_specs=pl.BlockSpec((BM, C), lambda i: (i, 0)),
        out_shape=jax.ShapeDtypeStruct((N, C), jnp.float32),
        scratch_shapes=[pltpu.VMEM((N, C), jnp.bfloat16)],
        compiler_params=pltpu.CompilerParams(
            dimension_semantics=("arbitrary",),
        ),
    )(y2, mask2, prop)
